# Initial kernel scaffold; baseline (speedup 1.0000x reference)
import jax
import jax.numpy as jnp
from jax.experimental import pallas as pl


def kernel(pred, target):
    raise NotImplementedError("write your pallas kernel here")


# speedup vs baseline: 1.5501x; 1.5501x over previous
import jax
import jax.numpy as jnp
from jax.experimental import pallas as pl
from jax.experimental.pallas import tpu as pltpu

N = 16384
BP = 1024
TB = 2048
NP = N // BP
NT = N // TB
LOSS_WEIGHT = 1.0


def _nn_loss_kernel(pred_ref, tgt_ref, out_ref, smin_ref, sval_ref, acc_ref):
    ip = pl.program_id(0)
    it = pl.program_id(1)

    @pl.when(it == 0)
    def _init():
        smin_ref[...] = jnp.full((BP, 1), jnp.inf, jnp.float32)
        sval_ref[...] = jnp.zeros((BP, 1), jnp.float32)

    pred_blk = pred_ref[...]
    pc = pred_blk[:, :3]
    tgt_blk = tgt_ref[...]
    tc = tgt_blk[:3, :]
    t_int = tgt_blk[3:4, :]

    tn = jnp.sum(tc * tc, axis=0, keepdims=True)
    dots = jax.lax.dot_general(
        pc, tc, (((1,), (0,)), ((), ())),
        preferred_element_type=jnp.float32)
    s = tn - 2.0 * dots

    m = jnp.min(s, axis=1, keepdims=True)
    v = jnp.min(jnp.where(s == m, t_int, jnp.inf),
                axis=1, keepdims=True)

    take = m < smin_ref[...]
    new_min = jnp.where(take, m, smin_ref[...])
    new_val = jnp.where(take, v, sval_ref[...])
    smin_ref[...] = new_min
    sval_ref[...] = new_val

    @pl.when(it == NT - 1)
    def _finish():
        p_int = pred_blk[:, 3:4]
        diff = p_int - new_val
        part = jnp.sum(diff * diff)

        @pl.when(ip == 0)
        def _():
            acc_ref[0, 0] = 0.0
        acc_ref[0, 0] += part

        @pl.when(ip == NP - 1)
        def _():
            out_ref[...] = jnp.full(
                (1, 1), acc_ref[0, 0] * (LOSS_WEIGHT / N), jnp.float32)


def kernel(pred, target):
    tgt_t = target.T
    out = pl.pallas_call(
        _nn_loss_kernel,
        grid=(NP, NT),
        in_specs=[
            pl.BlockSpec((BP, 4), lambda ip, it: (ip, 0)),
            pl.BlockSpec((4, TB), lambda ip, it: (0, it)),
        ],
        out_specs=pl.BlockSpec((1, 1), lambda ip, it: (0, 0)),
        out_shape=jax.ShapeDtypeStruct((1, 1), jnp.float32),
        scratch_shapes=[
            pltpu.VMEM((BP, 1), jnp.float32),
            pltpu.VMEM((BP, 1), jnp.float32),
            pltpu.SMEM((1, 1), jnp.float32),
        ],
        compiler_params=pltpu.CompilerParams(
            dimension_semantics=("arbitrary", "arbitrary")),
    )(pred, tgt_t)
    return jnp.reshape(out, ())


# speedup vs baseline: 1.7835x; 1.1505x over previous
import jax
import jax.numpy as jnp
from jax.experimental import pallas as pl
from jax.experimental.pallas import tpu as pltpu

N = 16384
BP = 1024
TB = 4096
NP = N // BP
NT = N // TB
LOSS_WEIGHT = 1.0


def _nn_loss_kernel(pred_ref, tgt_ref, out_ref, smin_ref, sval_ref, acc_ref):
    ip = pl.program_id(0)
    it = pl.program_id(1)

    @pl.when(it == 0)
    def _init():
        smin_ref[...] = jnp.full((BP, 1), jnp.inf, jnp.float32)
        sval_ref[...] = jnp.zeros((BP, 1), jnp.float32)

    pred_blk = pred_ref[...]
    tgt_blk = tgt_ref[...]
    t_int = tgt_blk[3:4, :]

    lane = jax.lax.broadcasted_iota(jnp.int32, (BP, 4), 1)
    laug = jnp.where(lane < 3, -2.0 * pred_blk, 1.0)
    sq = tgt_blk * tgt_blk
    tn = sq[0:1, :] + sq[1:2, :] + sq[2:3, :]
    row = jax.lax.broadcasted_iota(jnp.int32, (4, TB), 0)
    raug = jnp.where(row < 3, tgt_blk, tn)

    s = jax.lax.dot_general(
        laug, raug, (((1,), (0,)), ((), ())),
        preferred_element_type=jnp.float32)

    m = jnp.min(s, axis=1, keepdims=True)
    v = jnp.min(jnp.where(s == m, t_int, jnp.inf),
                axis=1, keepdims=True)

    take = m < smin_ref[...]
    new_val = jnp.where(take, v, sval_ref[...])
    smin_ref[...] = jnp.where(take, m, smin_ref[...])
    sval_ref[...] = new_val

    @pl.when(it == NT - 1)
    def _finish():
        p_int = pred_blk[:, 3:4]
        diff = p_int - new_val
        part = jnp.sum(diff * diff)

        @pl.when(ip == 0)
        def _():
            acc_ref[0, 0] = 0.0
        acc_ref[0, 0] += part

        @pl.when(ip == NP - 1)
        def _():
            out_ref[...] = jnp.full(
                (1, 1), acc_ref[0, 0] * (LOSS_WEIGHT / N), jnp.float32)


def kernel(pred, target):
    tgt_t = target.T
    out = pl.pallas_call(
        _nn_loss_kernel,
        grid=(NP, NT),
        in_specs=[
            pl.BlockSpec((BP, 4), lambda ip, it: (ip, 0)),
            pl.BlockSpec((4, TB), lambda ip, it: (0, it)),
        ],
        out_specs=pl.BlockSpec((1, 1), lambda ip, it: (0, 0)),
        out_shape=jax.ShapeDtypeStruct((1, 1), jnp.float32),
        scratch_shapes=[
            pltpu.VMEM((BP, 1), jnp.float32),
            pltpu.VMEM((BP, 1), jnp.float32),
            pltpu.SMEM((1, 1), jnp.float32),
        ],
        compiler_params=pltpu.CompilerParams(
            dimension_semantics=("arbitrary", "arbitrary")),
    )(pred, tgt_t)
    return jnp.reshape(out, ())
